# baseline (speedup 1.0000x reference)
import jax
import jax.numpy as jnp
from jax import lax
from jax.experimental import pallas as pl
from jax.experimental.pallas import tpu as pltpu
from jax.experimental.pallas import tpu_sc as plsc

_MAX_DET = 300
_OUTW = 304
_SCORE_THRESH = 0.3
_L = 16
_GS = 8
_SHARD = 640
_NSL = _SHARD // _L
_PAD = _SHARD + _L
_BIG = 1 << 30


def _sc_body(scal_hbm, x1_hbm, y1_hbm, x2_hbm, y2_hbm, sc_hbm, lb_hbm,
             out_hbm,
             scal_v, ox1_v, oy1_v, ox2_v, oy2_v,
             cx1_v, cy1_v, cx2_v, cy2_v, aj_v, sc_v, lb_v,
             alive_v, gi_v,
             rec_v, rb_v, o_v, pub_sh):
    c = lax.axis_index("c")
    s = lax.axis_index("s")
    grp = s // _GS
    g = s % _GS
    img = c * 2 + grp
    base = g * _SHARD
    lane = lax.iota(jnp.int32, _L)
    neg_inf = jnp.float32(-jnp.inf)
    ninf_vec = jnp.full((_L,), neg_inf, jnp.float32)
    basev = jnp.full((_L,), base, jnp.int32)

    pltpu.sync_copy(scal_hbm, scal_v)
    sv = scal_v[...]
    imgf = sv[0]
    thr = sv[1]
    cw0 = sv[2]
    cw1 = sv[3]
    pltpu.sync_copy(x1_hbm.at[img, pl.ds(base, _SHARD)], cx1_v)
    pltpu.sync_copy(y1_hbm.at[img, pl.ds(base, _SHARD)], cy1_v)
    pltpu.sync_copy(x2_hbm.at[img, pl.ds(base, _SHARD)], cx2_v)
    pltpu.sync_copy(y2_hbm.at[img, pl.ds(base, _SHARD)], cy2_v)
    pltpu.sync_copy(sc_hbm.at[img, pl.ds(base, _SHARD)], sc_v)
    pltpu.sync_copy(lb_hbm.at[img, pl.ds(base, _SHARD)], lb_v)

    def prep(si, w):
        d = pl.ds(si * _L, _L)
        lb = lb_v[d]
        lbf = lb.astype(jnp.float32)
        off = lbf * (imgf + 1.0)
        cx1 = jnp.minimum(jnp.maximum(cx1_v[d], 0.0), imgf)
        cy1 = jnp.minimum(jnp.maximum(cy1_v[d], 0.0), imgf)
        cx2 = jnp.minimum(jnp.maximum(cx2_v[d], 0.0), imgf)
        cy2 = jnp.minimum(jnp.maximum(cy2_v[d], 0.0), imgf)
        cx1_v[d] = cx1
        cy1_v[d] = cy1
        cx2_v[d] = cx2
        cy2_v[d] = cy2
        x1 = cx1 + off
        y1 = cy1 + off
        x2 = cx2 + off
        y2 = cy2 + off
        ox1_v[d] = x1
        oy1_v[d] = y1
        ox2_v[d] = x2
        oy2_v[d] = y2
        aj_v[d] = (x2 - x1) * (y2 - y1)
        sc = sc_v[d]
        w_sc = sc * jnp.where(lb == 0, cw0, cw1)
        valid = sc > _SCORE_THRESH
        plsc.store_compressed(alive_v.at[pl.ds(w, _L)], w_sc, mask=valid)
        plsc.store_compressed(gi_v.at[pl.ds(w, _L)],
                              lane + (base + si * _L), mask=valid)
        return w + plsc.all_reduce_population_count(valid)[0]

    n0 = lax.fori_loop(0, _NSL, prep, jnp.int32(0))
    alive_v[pl.ds(n0, _L)] = ninf_vec
    gi_v[pl.ds(n0, _L)] = basev

    @pl.when(g == 0)
    def _():
        z = jnp.where(lane == 7, jnp.float32(-1.0), jnp.float32(0.0))

        def zinit(si, _):
            o_v[si] = z
            return 0

        lax.fori_loop(0, _OUTW, zinit, 0)

    rec_v[...] = ninf_vec
    pltpu.sync_copy(rec_v, pub_sh.at[1, grp, g])
    pltpu.sync_copy(rec_v, pub_sh.at[2, grp, g])
    plsc.subcore_barrier()

    def step(t, carry):
        count, n = carry
        nv = (n + _L - 1) // _L

        def amax(si, c2):
            mv, mi = c2
            v = alive_v[pl.ds(si * _L, _L)]
            ii = lane + si * _L
            better = v > mv
            return jnp.where(better, v, mv), jnp.where(better, ii, mi)

        mv, mi = lax.fori_loop(
            0, nv, amax, (ninf_vec, jnp.zeros((_L,), jnp.int32)))
        m = jnp.max(mv)
        win = jnp.min(jnp.where(mv == m, mi, _BIG))
        winv = jnp.full((_L,), win, jnp.int32)
        gwi = plsc.load_gather(gi_v, [winv])
        olv = gwi - basev
        gx1 = plsc.load_gather(ox1_v, [olv])
        gy1 = plsc.load_gather(oy1_v, [olv])
        gx2 = plsc.load_gather(ox2_v, [olv])
        gy2 = plsc.load_gather(oy2_v, [olv])
        gcx1 = plsc.load_gather(cx1_v, [olv])
        gcy1 = plsc.load_gather(cy1_v, [olv])
        gcx2 = plsc.load_gather(cx2_v, [olv])
        gcy2 = plsc.load_gather(cy2_v, [olv])
        gsc = plsc.load_gather(sc_v, [olv])
        glb = plsc.load_gather(lb_v, [olv])
        rec = jnp.full((_L,), m, jnp.float32)
        rec = jnp.where(lane == 1, gwi.astype(jnp.float32), rec)
        rec = jnp.where(lane == 2, gx1, rec)
        rec = jnp.where(lane == 3, gy1, rec)
        rec = jnp.where(lane == 4, gx2, rec)
        rec = jnp.where(lane == 5, gy2, rec)
        rec = jnp.where(lane == 6, gsc, rec)
        rec = jnp.where(lane == 7, glb.astype(jnp.float32), rec)
        rec = jnp.where(lane == 8, gcx1, rec)
        rec = jnp.where(lane == 9, gcy1, rec)
        rec = jnp.where(lane == 10, gcx2, rec)
        rec = jnp.where(lane == 11, gcy2, rec)
        rec_v[...] = rec

        p = t % 2 + 1
        pltpu.sync_copy(rec_v, pub_sh.at[p, grp, g])
        plsc.subcore_barrier()
        pltpu.sync_copy(pub_sh.at[p, grp], rb_v)

        ok8 = lane < _GS
        zz = jnp.zeros((_L,), jnp.int32)
        val8 = plsc.load_gather(rb_v, [lane, zz], mask=ok8)
        val8 = jnp.where(ok8, val8, neg_inf)
        gid8 = plsc.load_gather(rb_v, [lane, zz + 1], mask=ok8)
        gid8 = jnp.where(ok8, gid8, jnp.float32(jnp.inf))
        m2 = jnp.max(val8)
        active = m2 > neg_inf
        wg = jnp.min(jnp.where(val8 == m2, gid8, jnp.float32(jnp.inf)))
        wk = jnp.min(jnp.where((val8 == m2) & (gid8 == wg), lane, _L))
        wkv = jnp.full((_L,), wk, jnp.int32)
        bx1 = plsc.load_gather(rb_v, [wkv, zz + 2])
        by1 = plsc.load_gather(rb_v, [wkv, zz + 3])
        bx2 = plsc.load_gather(rb_v, [wkv, zz + 4])
        by2 = plsc.load_gather(rb_v, [wkv, zz + 5])
        a_iv = (bx2 - bx1) * (by2 - by1)
        wgi = wg.astype(jnp.int32)
        wgiv = jnp.full((_L,), wgi, jnp.int32)

        nv2 = jnp.where(active, nv, 0)

        def supp(si, w):
            d = pl.ds(si * _L, _L)
            a = alive_v[d]
            gi = gi_v[d]
            ol = gi - basev
            x1 = plsc.load_gather(ox1_v, [ol])
            y1 = plsc.load_gather(oy1_v, [ol])
            x2 = plsc.load_gather(ox2_v, [ol])
            y2 = plsc.load_gather(oy2_v, [ol])
            aj = plsc.load_gather(aj_v, [ol])
            xx1 = jnp.maximum(bx1, x1)
            yy1 = jnp.maximum(by1, y1)
            xx2 = jnp.minimum(bx2, x2)
            yy2 = jnp.minimum(by2, y2)
            inter = (jnp.maximum(xx2 - xx1, 0.0)
                     * jnp.maximum(yy2 - yy1, 0.0))
            iou = inter / (a_iv + aj - inter + 1e-9)
            keep = jnp.logical_not((iou > thr) | (gi == wgiv)) & (a > neg_inf)
            plsc.store_compressed(alive_v.at[pl.ds(w, _L)], a, mask=keep)
            plsc.store_compressed(gi_v.at[pl.ds(w, _L)], gi, mask=keep)
            return w + plsc.all_reduce_population_count(keep)[0]

        nf = lax.fori_loop(0, nv2, supp, jnp.int32(0))

        @pl.when(active)
        def _():
            alive_v[pl.ds(nf, _L)] = ninf_vec
            gi_v[pl.ds(nf, _L)] = basev

        @pl.when(active & (g == 0) & (count < _MAX_DET))
        def _():
            o_v[count] = plsc.load_gather(rb_v, [wkv, lane])

        return (count + jnp.where(active, 1, 0), jnp.where(active, nf, n))

    lax.fori_loop(0, _MAX_DET + 2, step, (jnp.int32(0), n0))

    @pl.when(g == 0)
    def _():
        pltpu.sync_copy(o_v, out_hbm.at[img])


def kernel(boxes, scores, labels, img_size, nms_thresh, class_weights):
    B, N, _ = boxes.shape
    Np = _GS * _SHARD
    pad = Np - N

    thr = jax.nn.sigmoid(jnp.asarray(nms_thresh, jnp.float32))
    imgf = jnp.asarray(img_size, jnp.float32)
    cw = jnp.asarray(class_weights, jnp.float32)
    scal = jnp.zeros((16,), jnp.float32)
    scal = scal.at[0].set(imgf).at[1].set(thr)
    scal = scal.at[2].set(cw[0]).at[3].set(cw[1])

    x1 = jnp.pad(boxes[:, :, 0], ((0, 0), (0, pad)))
    y1 = jnp.pad(boxes[:, :, 1], ((0, 0), (0, pad)))
    x2 = jnp.pad(boxes[:, :, 2], ((0, 0), (0, pad)))
    y2 = jnp.pad(boxes[:, :, 3], ((0, 0), (0, pad)))
    scp = jnp.pad(scores, ((0, 0), (0, pad)), constant_values=-1.0)
    lbp = jnp.pad(labels.astype(jnp.int32), ((0, 0), (0, pad)))

    mesh = plsc.VectorSubcoreMesh(core_axis_name="c", subcore_axis_name="s",
                                  num_cores=2, num_subcores=16)
    run = pl.kernel(
        _sc_body,
        out_type=jax.ShapeDtypeStruct((B, _OUTW, _L), jnp.float32),
        mesh=mesh,
        compiler_params=pltpu.CompilerParams(needs_layout_passes=False),
        scratch_types=[
            pltpu.VMEM((_L,), jnp.float32),
            pltpu.VMEM((_SHARD,), jnp.float32),
            pltpu.VMEM((_SHARD,), jnp.float32),
            pltpu.VMEM((_SHARD,), jnp.float32),
            pltpu.VMEM((_SHARD,), jnp.float32),
            pltpu.VMEM((_SHARD,), jnp.float32),
            pltpu.VMEM((_SHARD,), jnp.float32),
            pltpu.VMEM((_SHARD,), jnp.float32),
            pltpu.VMEM((_SHARD,), jnp.float32),
            pltpu.VMEM((_SHARD,), jnp.float32),
            pltpu.VMEM((_SHARD,), jnp.float32),
            pltpu.VMEM((_SHARD,), jnp.int32),
            pltpu.VMEM((_PAD,), jnp.float32),
            pltpu.VMEM((_PAD,), jnp.int32),
            pltpu.VMEM((_L,), jnp.float32),
            pltpu.VMEM((_GS, _L), jnp.float32),
            pltpu.VMEM((_OUTW, _L), jnp.float32),
            pltpu.VMEM_SHARED((3, 2, _GS, _L), jnp.float32),
        ],
    )
    out = run(scal, x1, y1, x2, y2, scp, lbp)

    det = out[:, :_MAX_DET, :]
    out_boxes = det[:, :, 8:12]
    out_scores = det[:, :, 6]
    out_labels = det[:, :, 7].astype(jnp.int32)
    return out_boxes, out_scores, out_labels


# speedup vs baseline: 1.0339x; 1.0339x over previous
import jax
import jax.numpy as jnp
from jax import lax
from jax.experimental import pallas as pl
from jax.experimental.pallas import tpu as pltpu
from jax.experimental.pallas import tpu_sc as plsc

_MAX_DET = 300
_OUTW = 304
_SCORE_THRESH = 0.3
_L = 16
_GS = 8
_SHARD = 640
_NSL = _SHARD // _L
_BIG = 1 << 30


def _sc_body(scal_hbm, x1_hbm, y1_hbm, x2_hbm, y2_hbm, sc_hbm, lb_hbm,
             out_hbm,
             scal_v, ox1_v, oy1_v, ox2_v, oy2_v,
             cx1_v, cy1_v, cx2_v, cy2_v, aj_v, alive_v, sc_v, lb_v,
             rec_v, rb_v, o_v, pub_sh):
    c = lax.axis_index("c")
    s = lax.axis_index("s")
    grp = s // _GS
    g = s % _GS
    img = c * 2 + grp
    base = g * _SHARD
    lane = lax.iota(jnp.int32, _L)
    neg_inf = jnp.float32(-jnp.inf)
    pos_inf = jnp.float32(jnp.inf)
    ninf_vec = jnp.full((_L,), neg_inf, jnp.float32)
    pinf_vec = jnp.full((_L,), pos_inf, jnp.float32)

    pltpu.sync_copy(scal_hbm, scal_v)
    sv = scal_v[...]
    imgf = sv[0]
    thr = sv[1]
    cw0 = sv[2]
    cw1 = sv[3]
    pltpu.sync_copy(x1_hbm.at[img, pl.ds(base, _SHARD)], cx1_v)
    pltpu.sync_copy(y1_hbm.at[img, pl.ds(base, _SHARD)], cy1_v)
    pltpu.sync_copy(x2_hbm.at[img, pl.ds(base, _SHARD)], cx2_v)
    pltpu.sync_copy(y2_hbm.at[img, pl.ds(base, _SHARD)], cy2_v)
    pltpu.sync_copy(sc_hbm.at[img, pl.ds(base, _SHARD)], sc_v)
    pltpu.sync_copy(lb_hbm.at[img, pl.ds(base, _SHARD)], lb_v)

    def prep(si, c2):
        mv, mi = c2
        d = pl.ds(si * _L, _L)
        lb = lb_v[d]
        lbf = lb.astype(jnp.float32)
        off = lbf * (imgf + 1.0)
        cx1 = jnp.minimum(jnp.maximum(cx1_v[d], 0.0), imgf)
        cy1 = jnp.minimum(jnp.maximum(cy1_v[d], 0.0), imgf)
        cx2 = jnp.minimum(jnp.maximum(cx2_v[d], 0.0), imgf)
        cy2 = jnp.minimum(jnp.maximum(cy2_v[d], 0.0), imgf)
        cx1_v[d] = cx1
        cy1_v[d] = cy1
        cx2_v[d] = cx2
        cy2_v[d] = cy2
        x1 = cx1 + off
        y1 = cy1 + off
        x2 = cx2 + off
        y2 = cy2 + off
        ox1_v[d] = x1
        oy1_v[d] = y1
        ox2_v[d] = x2
        oy2_v[d] = y2
        aj_v[d] = (x2 - x1) * (y2 - y1)
        sc = sc_v[d]
        w = sc * jnp.where(lb == 0, cw0, cw1)
        a = jnp.where(sc > _SCORE_THRESH, w, neg_inf)
        alive_v[d] = a
        better = a > mv
        return (jnp.where(better, a, mv),
                jnp.where(better, lane + si * _L, mi))

    mv0, mi0 = lax.fori_loop(0, _NSL, prep,
                             (ninf_vec, jnp.zeros((_L,), jnp.int32)))

    @pl.when(g == 0)
    def _():
        z = jnp.where(lane == 7, jnp.float32(-1.0), jnp.float32(0.0))

        def zinit(si, _):
            o_v[si] = z
            return 0

        lax.fori_loop(0, _OUTW, zinit, 0)

    rec_v[...] = ninf_vec
    pltpu.sync_copy(rec_v, pub_sh.at[1, grp, g])
    pltpu.sync_copy(rec_v, pub_sh.at[2, grp, g])
    plsc.subcore_barrier()

    def step(t, carry):
        count, mv, mi = carry
        m = jnp.max(mv)
        win = jnp.min(jnp.where(mv == m, mi, _BIG))
        winv = jnp.full((_L,), win, jnp.int32)
        gx1 = plsc.load_gather(ox1_v, [winv])
        gy1 = plsc.load_gather(oy1_v, [winv])
        gx2 = plsc.load_gather(ox2_v, [winv])
        gy2 = plsc.load_gather(oy2_v, [winv])
        gcx1 = plsc.load_gather(cx1_v, [winv])
        gcy1 = plsc.load_gather(cy1_v, [winv])
        gcx2 = plsc.load_gather(cx2_v, [winv])
        gcy2 = plsc.load_gather(cy2_v, [winv])
        gsc = plsc.load_gather(sc_v, [winv])
        glb = plsc.load_gather(lb_v, [winv])
        rec = jnp.full((_L,), m, jnp.float32)
        rec = jnp.where(lane == 1, (base + win).astype(jnp.float32), rec)
        rec = jnp.where(lane == 2, gx1, rec)
        rec = jnp.where(lane == 3, gy1, rec)
        rec = jnp.where(lane == 4, gx2, rec)
        rec = jnp.where(lane == 5, gy2, rec)
        rec = jnp.where(lane == 6, gsc, rec)
        rec = jnp.where(lane == 7, glb.astype(jnp.float32), rec)
        rec = jnp.where(lane == 8, gcx1, rec)
        rec = jnp.where(lane == 9, gcy1, rec)
        rec = jnp.where(lane == 10, gcx2, rec)
        rec = jnp.where(lane == 11, gcy2, rec)
        rec_v[...] = rec

        p = t % 2 + 1
        pltpu.sync_copy(rec_v, pub_sh.at[p, grp, g])
        plsc.subcore_barrier()
        pltpu.sync_copy(pub_sh.at[p, grp], rb_v)

        ok8 = lane < _GS
        zz = jnp.zeros((_L,), jnp.int32)
        val8 = plsc.load_gather(rb_v, [lane, zz], mask=ok8)
        val8 = jnp.where(ok8, val8, neg_inf)
        gid8 = plsc.load_gather(rb_v, [lane, zz + 1], mask=ok8)
        gid8 = jnp.where(ok8, gid8, pos_inf)
        m2 = jnp.max(val8)
        active = m2 > neg_inf
        wg = jnp.min(jnp.where(val8 == m2, gid8, pos_inf))
        wk = jnp.min(jnp.where((val8 == m2) & (gid8 == wg), lane, _L))
        wkv = jnp.full((_L,), wk, jnp.int32)
        bx1 = jnp.where(active, plsc.load_gather(rb_v, [wkv, zz + 2]), pinf_vec)
        by1 = jnp.where(active, plsc.load_gather(rb_v, [wkv, zz + 3]), pinf_vec)
        bx2 = jnp.where(active, plsc.load_gather(rb_v, [wkv, zz + 4]), ninf_vec)
        by2 = jnp.where(active, plsc.load_gather(rb_v, [wkv, zz + 5]), ninf_vec)
        a_iv = (bx2 - bx1) * (by2 - by1)
        wgi = jnp.where(active, wg.astype(jnp.int32), -1)
        wgiv = jnp.full((_L,), wgi, jnp.int32)
        gbase = jnp.full((_L,), base, jnp.int32)

        def fused(si, c2):
            mv2, mi2 = c2
            d = pl.ds(si * _L, _L)
            a = alive_v[d]
            xx1 = jnp.maximum(bx1, ox1_v[d])
            yy1 = jnp.maximum(by1, oy1_v[d])
            xx2 = jnp.minimum(bx2, ox2_v[d])
            yy2 = jnp.minimum(by2, oy2_v[d])
            inter = (jnp.maximum(xx2 - xx1, 0.0)
                     * jnp.maximum(yy2 - yy1, 0.0))
            iou = inter / (a_iv + aj_v[d] - inter + 1e-9)
            ii = lane + si * _L
            kill = (iou > thr) | (ii + gbase == wgiv)
            a = jnp.where(kill, neg_inf, a)
            alive_v[d] = a
            better = a > mv2
            return (jnp.where(better, a, mv2),
                    jnp.where(better, ii, mi2))

        mvn, min_ = lax.fori_loop(
            0, _NSL, fused,
            (ninf_vec, jnp.zeros((_L,), jnp.int32)), unroll=2)

        @pl.when(active & (g == 0) & (count < _MAX_DET))
        def _():
            o_v[count] = plsc.load_gather(rb_v, [wkv, lane])

        return (count + jnp.where(active, 1, 0), mvn, min_)

    lax.fori_loop(0, _MAX_DET + 2, step,
                  (jnp.int32(0), mv0, mi0))

    @pl.when(g == 0)
    def _():
        pltpu.sync_copy(o_v, out_hbm.at[img])


def kernel(boxes, scores, labels, img_size, nms_thresh, class_weights):
    B, N, _ = boxes.shape
    Np = _GS * _SHARD
    pad = Np - N

    thr = jax.nn.sigmoid(jnp.asarray(nms_thresh, jnp.float32))
    imgf = jnp.asarray(img_size, jnp.float32)
    cw = jnp.asarray(class_weights, jnp.float32)
    scal = jnp.zeros((16,), jnp.float32)
    scal = scal.at[0].set(imgf).at[1].set(thr)
    scal = scal.at[2].set(cw[0]).at[3].set(cw[1])

    x1 = jnp.pad(boxes[:, :, 0], ((0, 0), (0, pad)))
    y1 = jnp.pad(boxes[:, :, 1], ((0, 0), (0, pad)))
    x2 = jnp.pad(boxes[:, :, 2], ((0, 0), (0, pad)))
    y2 = jnp.pad(boxes[:, :, 3], ((0, 0), (0, pad)))
    scp = jnp.pad(scores, ((0, 0), (0, pad)), constant_values=-1.0)
    lbp = jnp.pad(labels.astype(jnp.int32), ((0, 0), (0, pad)))

    mesh = plsc.VectorSubcoreMesh(core_axis_name="c", subcore_axis_name="s",
                                  num_cores=2, num_subcores=16)
    run = pl.kernel(
        _sc_body,
        out_type=jax.ShapeDtypeStruct((B, _OUTW, _L), jnp.float32),
        mesh=mesh,
        compiler_params=pltpu.CompilerParams(needs_layout_passes=False),
        scratch_types=[
            pltpu.VMEM((_L,), jnp.float32),
            pltpu.VMEM((_SHARD,), jnp.float32),
            pltpu.VMEM((_SHARD,), jnp.float32),
            pltpu.VMEM((_SHARD,), jnp.float32),
            pltpu.VMEM((_SHARD,), jnp.float32),
            pltpu.VMEM((_SHARD,), jnp.float32),
            pltpu.VMEM((_SHARD,), jnp.float32),
            pltpu.VMEM((_SHARD,), jnp.float32),
            pltpu.VMEM((_SHARD,), jnp.float32),
            pltpu.VMEM((_SHARD,), jnp.float32),
            pltpu.VMEM((_SHARD,), jnp.float32),
            pltpu.VMEM((_SHARD,), jnp.float32),
            pltpu.VMEM((_SHARD,), jnp.int32),
            pltpu.VMEM((_L,), jnp.float32),
            pltpu.VMEM((_GS, _L), jnp.float32),
            pltpu.VMEM((_OUTW, _L), jnp.float32),
            pltpu.VMEM_SHARED((3, 2, _GS, _L), jnp.float32),
        ],
    )
    out = run(scal, x1, y1, x2, y2, scp, lbp)

    det = out[:, :_MAX_DET, :]
    out_boxes = det[:, :, 8:12]
    out_scores = det[:, :, 6]
    out_labels = det[:, :, 7].astype(jnp.int32)
    return out_boxes, out_scores, out_labels


# speedup vs baseline: 1.9538x; 1.8897x over previous
import jax
import jax.numpy as jnp
from jax import lax
from jax.experimental import pallas as pl
from jax.experimental.pallas import tpu as pltpu
from jax.experimental.pallas import tpu_sc as plsc

_MAX_DET = 300
_OUTW = 304
_SCORE_THRESH = 0.3
_L = 16
_GS = 8
_SHARD = 640
_NSL = _SHARD // _L
_BIG = 1 << 30


def _sc_body(scal_hbm, x1_hbm, y1_hbm, x2_hbm, y2_hbm, sc_hbm, lb_hbm,
             out_hbm,
             scal_v, ox1_v, oy1_v, ox2_v, oy2_v,
             cx1_v, cy1_v, cx2_v, cy2_v, aj_v, alive_v, sc_v, lb_v,
             rec_v, rb_v, o_v, pub_sh):
    c = lax.axis_index("c")
    s = lax.axis_index("s")
    grp = s // _GS
    g = s % _GS
    img = c * 2 + grp
    base = g * _SHARD
    lane = lax.iota(jnp.int32, _L)
    neg_inf = jnp.float32(-jnp.inf)

    pltpu.sync_copy(scal_hbm, scal_v)
    sv = scal_v[...]
    imgf = sv[0]
    thr = sv[1]
    cw0 = sv[2]
    cw1 = sv[3]
    pltpu.sync_copy(x1_hbm.at[img, pl.ds(base, _SHARD)], cx1_v)
    pltpu.sync_copy(y1_hbm.at[img, pl.ds(base, _SHARD)], cy1_v)
    pltpu.sync_copy(x2_hbm.at[img, pl.ds(base, _SHARD)], cx2_v)
    pltpu.sync_copy(y2_hbm.at[img, pl.ds(base, _SHARD)], cy2_v)
    pltpu.sync_copy(sc_hbm.at[img, pl.ds(base, _SHARD)], sc_v)
    pltpu.sync_copy(lb_hbm.at[img, pl.ds(base, _SHARD)], lb_v)

    def prep(si, _):
        d = pl.ds(si * _L, _L)
        lb = lb_v[d]
        lbf = lb.astype(jnp.float32)
        off = lbf * (imgf + 1.0)
        cx1 = jnp.minimum(jnp.maximum(cx1_v[d], 0.0), imgf)
        cy1 = jnp.minimum(jnp.maximum(cy1_v[d], 0.0), imgf)
        cx2 = jnp.minimum(jnp.maximum(cx2_v[d], 0.0), imgf)
        cy2 = jnp.minimum(jnp.maximum(cy2_v[d], 0.0), imgf)
        cx1_v[d] = cx1
        cy1_v[d] = cy1
        cx2_v[d] = cx2
        cy2_v[d] = cy2
        x1 = cx1 + off
        y1 = cy1 + off
        x2 = cx2 + off
        y2 = cy2 + off
        ox1_v[d] = x1
        oy1_v[d] = y1
        ox2_v[d] = x2
        oy2_v[d] = y2
        aj_v[d] = (x2 - x1) * (y2 - y1)
        sc = sc_v[d]
        w = sc * jnp.where(lb == 0, cw0, cw1)
        alive_v[d] = jnp.where(sc > _SCORE_THRESH, w, neg_inf)
        return 0

    lax.fori_loop(0, _NSL, prep, 0)

    @pl.when(g == 0)
    def _():
        z = jnp.where(lane == 7, jnp.float32(-1.0), jnp.float32(0.0))

        def zinit(si, _):
            o_v[si] = z
            return 0

        lax.fori_loop(0, _OUTW, zinit, 0)

    rec_v[...] = jnp.full((_L,), neg_inf, jnp.float32)
    pltpu.sync_copy(rec_v, pub_sh.at[1, grp, g])
    pltpu.sync_copy(rec_v, pub_sh.at[2, grp, g])
    plsc.subcore_barrier()

    def step(t, count):
        def amax(si, carry):
            mv, mi = carry
            v = alive_v[pl.ds(si * _L, _L)]
            ii = lane + si * _L
            better = v > mv
            return jnp.where(better, v, mv), jnp.where(better, ii, mi)

        mv, mi = plsc.parallel_loop(
            0, _NSL, unroll=4,
            carry=(jnp.full((_L,), neg_inf, jnp.float32),
                   jnp.zeros((_L,), jnp.int32)))(amax)
        m = jnp.max(mv)
        win = jnp.min(jnp.where(mv == m, mi, _BIG))
        winv = jnp.full((_L,), win, jnp.int32)
        gx1 = plsc.load_gather(ox1_v, [winv])
        gy1 = plsc.load_gather(oy1_v, [winv])
        gx2 = plsc.load_gather(ox2_v, [winv])
        gy2 = plsc.load_gather(oy2_v, [winv])
        gcx1 = plsc.load_gather(cx1_v, [winv])
        gcy1 = plsc.load_gather(cy1_v, [winv])
        gcx2 = plsc.load_gather(cx2_v, [winv])
        gcy2 = plsc.load_gather(cy2_v, [winv])
        gsc = plsc.load_gather(sc_v, [winv])
        glb = plsc.load_gather(lb_v, [winv])
        rec = jnp.full((_L,), m, jnp.float32)
        rec = jnp.where(lane == 1, (base + win).astype(jnp.float32), rec)
        rec = jnp.where(lane == 2, gx1, rec)
        rec = jnp.where(lane == 3, gy1, rec)
        rec = jnp.where(lane == 4, gx2, rec)
        rec = jnp.where(lane == 5, gy2, rec)
        rec = jnp.where(lane == 6, gsc, rec)
        rec = jnp.where(lane == 7, glb.astype(jnp.float32), rec)
        rec = jnp.where(lane == 8, gcx1, rec)
        rec = jnp.where(lane == 9, gcy1, rec)
        rec = jnp.where(lane == 10, gcx2, rec)
        rec = jnp.where(lane == 11, gcy2, rec)
        rec_v[...] = rec

        p = t % 2 + 1
        pltpu.sync_copy(rec_v, pub_sh.at[p, grp, g])
        plsc.subcore_barrier()
        pltpu.sync_copy(pub_sh.at[p, grp], rb_v)

        ok8 = lane < _GS
        zz = jnp.zeros((_L,), jnp.int32)
        val8 = plsc.load_gather(rb_v, [lane, zz], mask=ok8)
        val8 = jnp.where(ok8, val8, neg_inf)
        gid8 = plsc.load_gather(rb_v, [lane, zz + 1], mask=ok8)
        gid8 = jnp.where(ok8, gid8, jnp.float32(jnp.inf))
        m2 = jnp.max(val8)
        active = m2 > neg_inf
        wg = jnp.min(jnp.where(val8 == m2, gid8, jnp.float32(jnp.inf)))
        wk = jnp.min(jnp.where((val8 == m2) & (gid8 == wg), lane, _L))
        wkv = jnp.full((_L,), wk, jnp.int32)
        bx1 = plsc.load_gather(rb_v, [wkv, zz + 2])
        by1 = plsc.load_gather(rb_v, [wkv, zz + 3])
        bx2 = plsc.load_gather(rb_v, [wkv, zz + 4])
        by2 = plsc.load_gather(rb_v, [wkv, zz + 5])
        a_iv = (bx2 - bx1) * (by2 - by1)
        wgi = wg.astype(jnp.int32)

        @pl.when(active)
        def _():
            def supp(si):
                d = pl.ds(si * _L, _L)
                a = alive_v[d]
                xx1 = jnp.maximum(bx1, ox1_v[d])
                yy1 = jnp.maximum(by1, oy1_v[d])
                xx2 = jnp.minimum(bx2, ox2_v[d])
                yy2 = jnp.minimum(by2, oy2_v[d])
                inter = (jnp.maximum(xx2 - xx1, 0.0)
                         * jnp.maximum(yy2 - yy1, 0.0))
                iou = inter / (a_iv + aj_v[d] - inter + 1e-9)
                gidx = lane + (base + si * _L)
                kill = (iou > thr) | (gidx == wgi)
                alive_v[d] = jnp.where(kill, neg_inf, a)

            plsc.parallel_loop(0, _NSL, unroll=4)(supp)

        @pl.when(active & (g == 0) & (count < _MAX_DET))
        def _():
            o_v[count] = plsc.load_gather(rb_v, [wkv, lane])

        return count + jnp.where(active, 1, 0)

    lax.fori_loop(0, _MAX_DET + 2, step, jnp.int32(0))

    @pl.when(g == 0)
    def _():
        pltpu.sync_copy(o_v, out_hbm.at[img])


def kernel(boxes, scores, labels, img_size, nms_thresh, class_weights):
    B, N, _ = boxes.shape
    Np = _GS * _SHARD
    pad = Np - N

    thr = jax.nn.sigmoid(jnp.asarray(nms_thresh, jnp.float32))
    imgf = jnp.asarray(img_size, jnp.float32)
    cw = jnp.asarray(class_weights, jnp.float32)
    scal = jnp.zeros((16,), jnp.float32)
    scal = scal.at[0].set(imgf).at[1].set(thr)
    scal = scal.at[2].set(cw[0]).at[3].set(cw[1])

    x1 = jnp.pad(boxes[:, :, 0], ((0, 0), (0, pad)))
    y1 = jnp.pad(boxes[:, :, 1], ((0, 0), (0, pad)))
    x2 = jnp.pad(boxes[:, :, 2], ((0, 0), (0, pad)))
    y2 = jnp.pad(boxes[:, :, 3], ((0, 0), (0, pad)))
    scp = jnp.pad(scores, ((0, 0), (0, pad)), constant_values=-1.0)
    lbp = jnp.pad(labels.astype(jnp.int32), ((0, 0), (0, pad)))

    mesh = plsc.VectorSubcoreMesh(core_axis_name="c", subcore_axis_name="s",
                                  num_cores=2, num_subcores=16)
    run = pl.kernel(
        _sc_body,
        out_type=jax.ShapeDtypeStruct((B, _OUTW, _L), jnp.float32),
        mesh=mesh,
        compiler_params=pltpu.CompilerParams(needs_layout_passes=False),
        scratch_types=[
            pltpu.VMEM((_L,), jnp.float32),
            pltpu.VMEM((_SHARD,), jnp.float32),
            pltpu.VMEM((_SHARD,), jnp.float32),
            pltpu.VMEM((_SHARD,), jnp.float32),
            pltpu.VMEM((_SHARD,), jnp.float32),
            pltpu.VMEM((_SHARD,), jnp.float32),
            pltpu.VMEM((_SHARD,), jnp.float32),
            pltpu.VMEM((_SHARD,), jnp.float32),
            pltpu.VMEM((_SHARD,), jnp.float32),
            pltpu.VMEM((_SHARD,), jnp.float32),
            pltpu.VMEM((_SHARD,), jnp.float32),
            pltpu.VMEM((_SHARD,), jnp.float32),
            pltpu.VMEM((_SHARD,), jnp.int32),
            pltpu.VMEM((_L,), jnp.float32),
            pltpu.VMEM((_GS, _L), jnp.float32),
            pltpu.VMEM((_OUTW, _L), jnp.float32),
            pltpu.VMEM_SHARED((3, 2, _GS, _L), jnp.float32),
        ],
    )
    out = run(scal, x1, y1, x2, y2, scp, lbp)

    det = out[:, :_MAX_DET, :]
    out_boxes = det[:, :, 8:12]
    out_scores = det[:, :, 6]
    out_labels = det[:, :, 7].astype(jnp.int32)
    return out_boxes, out_scores, out_labels


# speedup vs baseline: 2.0220x; 1.0349x over previous
import jax
import jax.numpy as jnp
from jax import lax
from jax.experimental import pallas as pl
from jax.experimental.pallas import tpu as pltpu
from jax.experimental.pallas import tpu_sc as plsc

_MAX_DET = 300
_OUTW = 304
_SCORE_THRESH = 0.3
_L = 16
_GS = 8
_SHARD = 640
_NSL = _SHARD // _L
_BIG = 1 << 30


def _sc_body(scal_hbm, x1_hbm, y1_hbm, x2_hbm, y2_hbm, sc_hbm, lb_hbm,
             out_hbm,
             scal_v, ox1_v, oy1_v, ox2_v, oy2_v,
             cx1_v, cy1_v, cx2_v, cy2_v, aj_v, alive_v, sc_v, lb_v,
             rec_v, rb_v, o_v, pub_sh):
    c = lax.axis_index("c")
    s = lax.axis_index("s")
    grp = s // _GS
    g = s % _GS
    img = c * 2 + grp
    base = g * _SHARD
    lane = lax.iota(jnp.int32, _L)
    neg_inf = jnp.float32(-jnp.inf)

    pltpu.sync_copy(scal_hbm, scal_v)
    sv = scal_v[...]
    imgf = sv[0]
    thr = sv[1]
    cw0 = sv[2]
    cw1 = sv[3]
    pltpu.sync_copy(x1_hbm.at[img, pl.ds(base, _SHARD)], cx1_v)
    pltpu.sync_copy(y1_hbm.at[img, pl.ds(base, _SHARD)], cy1_v)
    pltpu.sync_copy(x2_hbm.at[img, pl.ds(base, _SHARD)], cx2_v)
    pltpu.sync_copy(y2_hbm.at[img, pl.ds(base, _SHARD)], cy2_v)
    pltpu.sync_copy(sc_hbm.at[img, pl.ds(base, _SHARD)], sc_v)
    pltpu.sync_copy(lb_hbm.at[img, pl.ds(base, _SHARD)], lb_v)

    def prep(si, c2):
        mv, mi = c2
        d = pl.ds(si * _L, _L)
        lb = lb_v[d]
        lbf = lb.astype(jnp.float32)
        off = lbf * (imgf + 1.0)
        cx1 = jnp.minimum(jnp.maximum(cx1_v[d], 0.0), imgf)
        cy1 = jnp.minimum(jnp.maximum(cy1_v[d], 0.0), imgf)
        cx2 = jnp.minimum(jnp.maximum(cx2_v[d], 0.0), imgf)
        cy2 = jnp.minimum(jnp.maximum(cy2_v[d], 0.0), imgf)
        cx1_v[d] = cx1
        cy1_v[d] = cy1
        cx2_v[d] = cx2
        cy2_v[d] = cy2
        x1 = cx1 + off
        y1 = cy1 + off
        x2 = cx2 + off
        y2 = cy2 + off
        ox1_v[d] = x1
        oy1_v[d] = y1
        ox2_v[d] = x2
        oy2_v[d] = y2
        aj_v[d] = (x2 - x1) * (y2 - y1)
        sc = sc_v[d]
        w = sc * jnp.where(lb == 0, cw0, cw1)
        a = jnp.where(sc > _SCORE_THRESH, w, neg_inf)
        alive_v[d] = a
        better = a > mv
        return (jnp.where(better, a, mv),
                jnp.where(better, lane + si * _L, mi))

    mv0, mi0 = lax.fori_loop(
        0, _NSL, prep,
        (jnp.full((_L,), neg_inf, jnp.float32), jnp.zeros((_L,), jnp.int32)))

    @pl.when(g == 0)
    def _():
        z = jnp.where(lane == 7, jnp.float32(-1.0), jnp.float32(0.0))

        def zinit(si, _):
            o_v[si] = z
            return 0

        lax.fori_loop(0, _OUTW, zinit, 0)

    rec_v[...] = jnp.full((_L,), neg_inf, jnp.float32)
    pltpu.sync_copy(rec_v, pub_sh.at[1, grp, g])
    pltpu.sync_copy(rec_v, pub_sh.at[2, grp, g])
    plsc.subcore_barrier()

    def step(t, carry):
        count, mv, mi = carry
        m = jnp.max(mv)
        win = jnp.min(jnp.where(mv == m, mi, _BIG))
        winv = jnp.full((_L,), win, jnp.int32)
        gx1 = plsc.load_gather(ox1_v, [winv])
        gy1 = plsc.load_gather(oy1_v, [winv])
        gx2 = plsc.load_gather(ox2_v, [winv])
        gy2 = plsc.load_gather(oy2_v, [winv])
        gcx1 = plsc.load_gather(cx1_v, [winv])
        gcy1 = plsc.load_gather(cy1_v, [winv])
        gcx2 = plsc.load_gather(cx2_v, [winv])
        gcy2 = plsc.load_gather(cy2_v, [winv])
        gsc = plsc.load_gather(sc_v, [winv])
        glb = plsc.load_gather(lb_v, [winv])
        rec = jnp.full((_L,), m, jnp.float32)
        rec = jnp.where(lane == 1, (base + win).astype(jnp.float32), rec)
        rec = jnp.where(lane == 2, gx1, rec)
        rec = jnp.where(lane == 3, gy1, rec)
        rec = jnp.where(lane == 4, gx2, rec)
        rec = jnp.where(lane == 5, gy2, rec)
        rec = jnp.where(lane == 6, gsc, rec)
        rec = jnp.where(lane == 7, glb.astype(jnp.float32), rec)
        rec = jnp.where(lane == 8, gcx1, rec)
        rec = jnp.where(lane == 9, gcy1, rec)
        rec = jnp.where(lane == 10, gcx2, rec)
        rec = jnp.where(lane == 11, gcy2, rec)
        rec_v[...] = rec

        p = t % 2 + 1
        pltpu.sync_copy(rec_v, pub_sh.at[p, grp, g])
        plsc.subcore_barrier()
        pltpu.sync_copy(pub_sh.at[p, grp], rb_v)

        ok8 = lane < _GS
        zz = jnp.zeros((_L,), jnp.int32)
        val8 = plsc.load_gather(rb_v, [lane, zz], mask=ok8)
        val8 = jnp.where(ok8, val8, neg_inf)
        gid8 = plsc.load_gather(rb_v, [lane, zz + 1], mask=ok8)
        gid8 = jnp.where(ok8, gid8, jnp.float32(jnp.inf))
        m2 = jnp.max(val8)
        active = m2 > neg_inf
        wg = jnp.min(jnp.where(val8 == m2, gid8, jnp.float32(jnp.inf)))
        wk = jnp.min(jnp.where((val8 == m2) & (gid8 == wg), lane, _L))
        wkv = jnp.full((_L,), wk, jnp.int32)
        pinf_vec = jnp.full((_L,), jnp.float32(jnp.inf), jnp.float32)
        ninf_vec = jnp.full((_L,), neg_inf, jnp.float32)
        bx1 = jnp.where(active, plsc.load_gather(rb_v, [wkv, zz + 2]), pinf_vec)
        by1 = jnp.where(active, plsc.load_gather(rb_v, [wkv, zz + 3]), pinf_vec)
        bx2 = jnp.where(active, plsc.load_gather(rb_v, [wkv, zz + 4]), ninf_vec)
        by2 = jnp.where(active, plsc.load_gather(rb_v, [wkv, zz + 5]), ninf_vec)
        a_iv = (bx2 - bx1) * (by2 - by1)
        wgi = jnp.where(active, wg.astype(jnp.int32), -1)

        def fused(si, c2):
            mv2, mi2 = c2
            d = pl.ds(si * _L, _L)
            a = alive_v[d]
            xx1 = jnp.maximum(bx1, ox1_v[d])
            yy1 = jnp.maximum(by1, oy1_v[d])
            xx2 = jnp.minimum(bx2, ox2_v[d])
            yy2 = jnp.minimum(by2, oy2_v[d])
            inter = (jnp.maximum(xx2 - xx1, 0.0)
                     * jnp.maximum(yy2 - yy1, 0.0))
            iou = inter / (a_iv + aj_v[d] - inter + 1e-9)
            ii = lane + si * _L
            kill = (iou > thr) | (ii + base == wgi)
            a = jnp.where(kill, neg_inf, a)
            alive_v[d] = a
            better = a > mv2
            return (jnp.where(better, a, mv2), jnp.where(better, ii, mi2))

        mvn, min_ = plsc.parallel_loop(
            0, _NSL, unroll=4,
            carry=(ninf_vec, jnp.zeros((_L,), jnp.int32)))(fused)

        @pl.when(active & (g == 0) & (count < _MAX_DET))
        def _():
            o_v[count] = plsc.load_gather(rb_v, [wkv, lane])

        return (count + jnp.where(active, 1, 0), mvn, min_)

    lax.fori_loop(0, _MAX_DET + 2, step, (jnp.int32(0), mv0, mi0))

    @pl.when(g == 0)
    def _():
        pltpu.sync_copy(o_v, out_hbm.at[img])


def kernel(boxes, scores, labels, img_size, nms_thresh, class_weights):
    B, N, _ = boxes.shape
    Np = _GS * _SHARD
    pad = Np - N

    thr = jax.nn.sigmoid(jnp.asarray(nms_thresh, jnp.float32))
    imgf = jnp.asarray(img_size, jnp.float32)
    cw = jnp.asarray(class_weights, jnp.float32)
    scal = jnp.zeros((16,), jnp.float32)
    scal = scal.at[0].set(imgf).at[1].set(thr)
    scal = scal.at[2].set(cw[0]).at[3].set(cw[1])

    x1 = jnp.pad(boxes[:, :, 0], ((0, 0), (0, pad)))
    y1 = jnp.pad(boxes[:, :, 1], ((0, 0), (0, pad)))
    x2 = jnp.pad(boxes[:, :, 2], ((0, 0), (0, pad)))
    y2 = jnp.pad(boxes[:, :, 3], ((0, 0), (0, pad)))
    scp = jnp.pad(scores, ((0, 0), (0, pad)), constant_values=-1.0)
    lbp = jnp.pad(labels.astype(jnp.int32), ((0, 0), (0, pad)))

    mesh = plsc.VectorSubcoreMesh(core_axis_name="c", subcore_axis_name="s",
                                  num_cores=2, num_subcores=16)
    run = pl.kernel(
        _sc_body,
        out_type=jax.ShapeDtypeStruct((B, _OUTW, _L), jnp.float32),
        mesh=mesh,
        compiler_params=pltpu.CompilerParams(needs_layout_passes=False),
        scratch_types=[
            pltpu.VMEM((_L,), jnp.float32),
            pltpu.VMEM((_SHARD,), jnp.float32),
            pltpu.VMEM((_SHARD,), jnp.float32),
            pltpu.VMEM((_SHARD,), jnp.float32),
            pltpu.VMEM((_SHARD,), jnp.float32),
            pltpu.VMEM((_SHARD,), jnp.float32),
            pltpu.VMEM((_SHARD,), jnp.float32),
            pltpu.VMEM((_SHARD,), jnp.float32),
            pltpu.VMEM((_SHARD,), jnp.float32),
            pltpu.VMEM((_SHARD,), jnp.float32),
            pltpu.VMEM((_SHARD,), jnp.float32),
            pltpu.VMEM((_SHARD,), jnp.float32),
            pltpu.VMEM((_SHARD,), jnp.int32),
            pltpu.VMEM((_L,), jnp.float32),
            pltpu.VMEM((_GS, _L), jnp.float32),
            pltpu.VMEM((_OUTW, _L), jnp.float32),
            pltpu.VMEM_SHARED((3, 2, _GS, _L), jnp.float32),
        ],
    )
    out = run(scal, x1, y1, x2, y2, scp, lbp)

    det = out[:, :_MAX_DET, :]
    out_boxes = det[:, :, 8:12]
    out_scores = det[:, :, 6]
    out_labels = det[:, :, 7].astype(jnp.int32)
    return out_boxes, out_scores, out_labels


# speedup vs baseline: 2.0573x; 1.0175x over previous
import jax
import jax.numpy as jnp
from jax import lax
from jax.experimental import pallas as pl
from jax.experimental.pallas import tpu as pltpu
from jax.experimental.pallas import tpu_sc as plsc

_MAX_DET = 300
_OUTW = 304
_SCORE_THRESH = 0.3
_L = 16
_GS = 8
_SHARD = 640
_NSL = _SHARD // _L
_BIG = 1 << 30


def _sc_body(scal_hbm, x1_hbm, y1_hbm, x2_hbm, y2_hbm, sc_hbm, lb_hbm,
             out_hbm,
             scal_v, ox1_v, oy1_v, ox2_v, oy2_v,
             cx1_v, cy1_v, cx2_v, cy2_v, aj_v, alive_v, sc_v, lb_v,
             rec_v, rb_v, o_v, pub_sh):
    c = lax.axis_index("c")
    s = lax.axis_index("s")
    grp = s // _GS
    g = s % _GS
    img = c * 2 + grp
    base = g * _SHARD
    lane = lax.iota(jnp.int32, _L)
    neg_inf = jnp.float32(-jnp.inf)

    pltpu.sync_copy(scal_hbm, scal_v)
    sv = scal_v[...]
    imgf = sv[0]
    thr = sv[1]
    cw0 = sv[2]
    cw1 = sv[3]
    pltpu.sync_copy(x1_hbm.at[img, pl.ds(base, _SHARD)], cx1_v)
    pltpu.sync_copy(y1_hbm.at[img, pl.ds(base, _SHARD)], cy1_v)
    pltpu.sync_copy(x2_hbm.at[img, pl.ds(base, _SHARD)], cx2_v)
    pltpu.sync_copy(y2_hbm.at[img, pl.ds(base, _SHARD)], cy2_v)
    pltpu.sync_copy(sc_hbm.at[img, pl.ds(base, _SHARD)], sc_v)
    pltpu.sync_copy(lb_hbm.at[img, pl.ds(base, _SHARD)], lb_v)

    def prep(si, c2):
        mv, mi = c2
        d = pl.ds(si * _L, _L)
        lb = lb_v[d]
        lbf = lb.astype(jnp.float32)
        off = lbf * (imgf + 1.0)
        cx1 = jnp.minimum(jnp.maximum(cx1_v[d], 0.0), imgf)
        cy1 = jnp.minimum(jnp.maximum(cy1_v[d], 0.0), imgf)
        cx2 = jnp.minimum(jnp.maximum(cx2_v[d], 0.0), imgf)
        cy2 = jnp.minimum(jnp.maximum(cy2_v[d], 0.0), imgf)
        cx1_v[d] = cx1
        cy1_v[d] = cy1
        cx2_v[d] = cx2
        cy2_v[d] = cy2
        x1 = cx1 + off
        y1 = cy1 + off
        x2 = cx2 + off
        y2 = cy2 + off
        ox1_v[d] = x1
        oy1_v[d] = y1
        ox2_v[d] = x2
        oy2_v[d] = y2
        aj_v[d] = (x2 - x1) * (y2 - y1)
        sc = sc_v[d]
        w = sc * jnp.where(lb == 0, cw0, cw1)
        a = jnp.where(sc > _SCORE_THRESH, w, neg_inf)
        alive_v[d] = a
        better = a > mv
        return (jnp.where(better, a, mv),
                jnp.where(better, lane + si * _L, mi))

    mv0, mi0 = lax.fori_loop(
        0, _NSL, prep,
        (jnp.full((_L,), neg_inf, jnp.float32), jnp.zeros((_L,), jnp.int32)))

    @pl.when(g == 0)
    def _():
        z = jnp.where(lane == 7, jnp.float32(-1.0), jnp.float32(0.0))

        def zinit(si, _):
            o_v[si] = z
            return 0

        lax.fori_loop(0, _OUTW, zinit, 0)

    rec_v[...] = jnp.full((_L,), neg_inf, jnp.float32)
    pltpu.sync_copy(rec_v, pub_sh.at[1, grp, g])
    pltpu.sync_copy(rec_v, pub_sh.at[2, grp, g])
    plsc.subcore_barrier()

    def step(t, carry):
        count, mv, mi = carry
        m = jnp.max(mv)
        win = jnp.min(jnp.where(mv == m, mi, _BIG))
        winv = jnp.full((_L,), win, jnp.int32)
        gx1 = plsc.load_gather(ox1_v, [winv])
        gy1 = plsc.load_gather(oy1_v, [winv])
        gx2 = plsc.load_gather(ox2_v, [winv])
        gy2 = plsc.load_gather(oy2_v, [winv])
        gcx1 = plsc.load_gather(cx1_v, [winv])
        gcy1 = plsc.load_gather(cy1_v, [winv])
        gcx2 = plsc.load_gather(cx2_v, [winv])
        gcy2 = plsc.load_gather(cy2_v, [winv])
        gsc = plsc.load_gather(sc_v, [winv])
        glb = plsc.load_gather(lb_v, [winv])
        rec = jnp.full((_L,), m, jnp.float32)
        rec = jnp.where(lane == 1, (base + win).astype(jnp.float32), rec)
        rec = jnp.where(lane == 2, gx1, rec)
        rec = jnp.where(lane == 3, gy1, rec)
        rec = jnp.where(lane == 4, gx2, rec)
        rec = jnp.where(lane == 5, gy2, rec)
        rec = jnp.where(lane == 6, gsc, rec)
        rec = jnp.where(lane == 7, glb.astype(jnp.float32), rec)
        rec = jnp.where(lane == 8, gcx1, rec)
        rec = jnp.where(lane == 9, gcy1, rec)
        rec = jnp.where(lane == 10, gcx2, rec)
        rec = jnp.where(lane == 11, gcy2, rec)
        rec_v[...] = rec

        p = t % 2 + 1
        pltpu.sync_copy(rec_v, pub_sh.at[p, grp, g])
        plsc.subcore_barrier()
        pltpu.sync_copy(pub_sh.at[p, grp], rb_v)

        ok8 = lane < _GS
        zz = jnp.zeros((_L,), jnp.int32)
        val8 = plsc.load_gather(rb_v, [lane, zz], mask=ok8)
        val8 = jnp.where(ok8, val8, neg_inf)
        gid8 = plsc.load_gather(rb_v, [lane, zz + 1], mask=ok8)
        gid8 = jnp.where(ok8, gid8, jnp.float32(jnp.inf))
        m2 = jnp.max(val8)
        active = m2 > neg_inf
        wg = jnp.min(jnp.where(val8 == m2, gid8, jnp.float32(jnp.inf)))
        wk = jnp.min(jnp.where((val8 == m2) & (gid8 == wg), lane, _L))
        wkv = jnp.full((_L,), wk, jnp.int32)
        pinf_vec = jnp.full((_L,), jnp.float32(jnp.inf), jnp.float32)
        ninf_vec = jnp.full((_L,), neg_inf, jnp.float32)
        bx1 = jnp.where(active, plsc.load_gather(rb_v, [wkv, zz + 2]), pinf_vec)
        by1 = jnp.where(active, plsc.load_gather(rb_v, [wkv, zz + 3]), pinf_vec)
        bx2 = jnp.where(active, plsc.load_gather(rb_v, [wkv, zz + 4]), ninf_vec)
        by2 = jnp.where(active, plsc.load_gather(rb_v, [wkv, zz + 5]), ninf_vec)
        a_iv = (bx2 - bx1) * (by2 - by1)
        wgi = jnp.where(active, wg.astype(jnp.int32), -1)

        def fused(si, c2):
            mv2, mi2 = c2
            d = pl.ds(si * _L, _L)
            a = alive_v[d]
            xx1 = jnp.maximum(bx1, ox1_v[d])
            yy1 = jnp.maximum(by1, oy1_v[d])
            xx2 = jnp.minimum(bx2, ox2_v[d])
            yy2 = jnp.minimum(by2, oy2_v[d])
            inter = (jnp.maximum(xx2 - xx1, 0.0)
                     * jnp.maximum(yy2 - yy1, 0.0))
            iou = inter / (a_iv + aj_v[d] - inter + 1e-9)
            ii = lane + si * _L
            kill = (iou > thr) | (ii + base == wgi)
            a = jnp.where(kill, neg_inf, a)
            alive_v[d] = a
            better = a > mv2
            return (jnp.where(better, a, mv2), jnp.where(better, ii, mi2))

        mvn, min_ = plsc.parallel_loop(
            0, _NSL, unroll=8,
            carry=(ninf_vec, jnp.zeros((_L,), jnp.int32)))(fused)

        @pl.when(active & (g == 0) & (count < _MAX_DET))
        def _():
            o_v[count] = plsc.load_gather(rb_v, [wkv, lane])

        return (count + jnp.where(active, 1, 0), mvn, min_)

    lax.fori_loop(0, _MAX_DET + 2, step, (jnp.int32(0), mv0, mi0))

    @pl.when(g == 0)
    def _():
        pltpu.sync_copy(o_v, out_hbm.at[img])


def kernel(boxes, scores, labels, img_size, nms_thresh, class_weights):
    B, N, _ = boxes.shape
    Np = _GS * _SHARD
    pad = Np - N

    thr = jax.nn.sigmoid(jnp.asarray(nms_thresh, jnp.float32))
    imgf = jnp.asarray(img_size, jnp.float32)
    cw = jnp.asarray(class_weights, jnp.float32)
    scal = jnp.zeros((16,), jnp.float32)
    scal = scal.at[0].set(imgf).at[1].set(thr)
    scal = scal.at[2].set(cw[0]).at[3].set(cw[1])

    x1 = jnp.pad(boxes[:, :, 0], ((0, 0), (0, pad)))
    y1 = jnp.pad(boxes[:, :, 1], ((0, 0), (0, pad)))
    x2 = jnp.pad(boxes[:, :, 2], ((0, 0), (0, pad)))
    y2 = jnp.pad(boxes[:, :, 3], ((0, 0), (0, pad)))
    scp = jnp.pad(scores, ((0, 0), (0, pad)), constant_values=-1.0)
    lbp = jnp.pad(labels.astype(jnp.int32), ((0, 0), (0, pad)))

    mesh = plsc.VectorSubcoreMesh(core_axis_name="c", subcore_axis_name="s",
                                  num_cores=2, num_subcores=16)
    run = pl.kernel(
        _sc_body,
        out_type=jax.ShapeDtypeStruct((B, _OUTW, _L), jnp.float32),
        mesh=mesh,
        compiler_params=pltpu.CompilerParams(needs_layout_passes=False),
        scratch_types=[
            pltpu.VMEM((_L,), jnp.float32),
            pltpu.VMEM((_SHARD,), jnp.float32),
            pltpu.VMEM((_SHARD,), jnp.float32),
            pltpu.VMEM((_SHARD,), jnp.float32),
            pltpu.VMEM((_SHARD,), jnp.float32),
            pltpu.VMEM((_SHARD,), jnp.float32),
            pltpu.VMEM((_SHARD,), jnp.float32),
            pltpu.VMEM((_SHARD,), jnp.float32),
            pltpu.VMEM((_SHARD,), jnp.float32),
            pltpu.VMEM((_SHARD,), jnp.float32),
            pltpu.VMEM((_SHARD,), jnp.float32),
            pltpu.VMEM((_SHARD,), jnp.float32),
            pltpu.VMEM((_SHARD,), jnp.int32),
            pltpu.VMEM((_L,), jnp.float32),
            pltpu.VMEM((_GS, _L), jnp.float32),
            pltpu.VMEM((_OUTW, _L), jnp.float32),
            pltpu.VMEM_SHARED((3, 2, _GS, _L), jnp.float32),
        ],
    )
    out = run(scal, x1, y1, x2, y2, scp, lbp)

    det = out[:, :_MAX_DET, :]
    out_boxes = det[:, :, 8:12]
    out_scores = det[:, :, 6]
    out_labels = det[:, :, 7].astype(jnp.int32)
    return out_boxes, out_scores, out_labels
